# baseline (speedup 1.0000x reference)
import functools

import jax
import jax.numpy as jnp
from jax import lax
from jax.experimental import pallas as pl
from jax.experimental.pallas import tpu as pltpu
from jax.experimental.pallas import tpu_sc as plsc

NC = 2
NS = 16
NW = NC * NS
NBUF = 8


def _make_emb(n_rows, seq, vocab, d):
    rows_per_w = n_rows // NW

    mesh = plsc.VectorSubcoreMesh(core_axis_name="c", subcore_axis_name="s")

    scratch = [
        pltpu.VMEM((rows_per_w, seq), jnp.int32),
        pltpu.VMEM((NBUF, seq, d), jnp.float32),
    ] + [pltpu.SemaphoreType.DMA] * (2 * NBUF)

    @functools.partial(
        pl.kernel,
        mesh=mesh,
        out_type=jax.ShapeDtypeStruct((n_rows, seq, d), jnp.float32),
        scratch_types=scratch,
        compiler_params=pltpu.CompilerParams(use_tc_tiling_on_sc=True),
    )
    def emb(x_hbm, w_hbm, out_hbm, idx_v, rows_v, *sems):
        gsems, osems = sems[:NBUF], sems[NBUF:]
        wid = lax.axis_index("s") * NC + lax.axis_index("c")
        base = wid * rows_per_w
        pltpu.sync_copy(x_hbm.at[pl.ds(base, rows_per_w)], idx_v)

        def gather(g, b):
            return pltpu.make_async_copy(
                w_hbm.at[idx_v.at[g]], rows_v.at[b], gsems[b]
            )

        def write(g, b):
            return pltpu.make_async_copy(rows_v.at[b], out_hbm.at[base + g], osems[b])

        def outer(go, carry):
            for b in range(NBUF):
                g = go * NBUF + b

                @pl.when(g >= NBUF)
                def _():
                    write(g - NBUF, b).wait()

                gather(g, b).start()

                gc = g - (NBUF - 1)
                bc = (b + 1) % NBUF

                @pl.when(gc >= 0)
                def _():
                    gather(gc, bc).wait()
                    write(gc, bc).start()

            return carry

        lax.fori_loop(0, rows_per_w // NBUF, outer, 0)

        for k in range(NBUF - 1):
            gc = rows_per_w - (NBUF - 1) + k
            gather(gc, gc % NBUF).wait()
            write(gc, gc % NBUF).start()
        for b in range(NBUF):
            write(rows_per_w - NBUF + b, b).wait()

    return emb


def kernel(x, weight):
    n_rows, seq = x.shape
    vocab, d = weight.shape
    return _make_emb(n_rows, seq, vocab, d)(x, weight)


# speedup vs baseline: 1.8007x; 1.8007x over previous
import functools

import jax
import jax.numpy as jnp
from jax import lax
from jax.experimental import pallas as pl
from jax.experimental.pallas import tpu as pltpu
from jax.experimental.pallas import tpu_sc as plsc

NC = 2
NS = 16
NW = NC * NS
NBUF = 5


def _make_emb(seq, n_rows, vocab, d):
    cols_per_w = n_rows // NW

    mesh = plsc.VectorSubcoreMesh(core_axis_name="c", subcore_axis_name="s")

    scratch = [
        pltpu.VMEM((seq, cols_per_w), jnp.int32),
        pltpu.VMEM((NBUF, cols_per_w, d), jnp.float32),
    ] + [pltpu.SemaphoreType.DMA] * (2 * NBUF)

    @functools.partial(
        pl.kernel,
        mesh=mesh,
        out_type=jax.ShapeDtypeStruct((seq, n_rows, d), jnp.float32),
        scratch_types=scratch,
    )
    def emb(xt_hbm, w_hbm, out_hbm, idx_v, rows_v, *sems):
        gsems, osems = sems[:NBUF], sems[NBUF:]
        wid = lax.axis_index("s") * NC + lax.axis_index("c")
        col0 = wid * cols_per_w
        pltpu.sync_copy(xt_hbm.at[:, pl.ds(col0, cols_per_w)], idx_v)

        def gather(j, b):
            return pltpu.make_async_copy(
                w_hbm.at[idx_v.at[j]], rows_v.at[b], gsems[b]
            )

        def write(j, b):
            return pltpu.make_async_copy(
                rows_v.at[b], out_hbm.at[j, pl.ds(col0, cols_per_w)], osems[b]
            )

        def outer(jo, carry):
            for b in range(NBUF):
                j = jo * NBUF + b

                @pl.when(j >= NBUF)
                def _():
                    write(j - NBUF, b).wait()

                gather(j, b).start()

                jc = j - (NBUF - 1)
                bc = (b + 1) % NBUF

                @pl.when(jc >= 0)
                def _():
                    gather(jc, bc).wait()
                    write(jc, bc).start()

            return carry

        lax.fori_loop(0, seq // NBUF, outer, 0)

        for k in range(NBUF - 1):
            jc = seq - (NBUF - 1) + k
            gather(jc, jc % NBUF).wait()
            write(jc, jc % NBUF).start()
        for b in range(NBUF):
            write(seq - NBUF + b, b).wait()

    return emb


def kernel(x, weight):
    n_rows, seq = x.shape
    vocab, d = weight.shape
    out_t = _make_emb(seq, n_rows, vocab, d)(x.T, weight)
    return out_t.transpose(1, 0, 2)


# speedup vs baseline: 1.8025x; 1.0010x over previous
import functools

import jax
import jax.numpy as jnp
from jax import lax
from jax.experimental import pallas as pl
from jax.experimental.pallas import tpu as pltpu
from jax.experimental.pallas import tpu_sc as plsc

NC = 2
NS = 16
NW = NC * NS
NBUF = 10
HSPLIT = 2


def _make_emb(seq, n_rows, vocab, d):
    cols_per_w = n_rows // NW
    hcols = cols_per_w // HSPLIT
    nchunk = seq * HSPLIT

    mesh = plsc.VectorSubcoreMesh(core_axis_name="c", subcore_axis_name="s")

    scratch = [
        pltpu.VMEM((seq, cols_per_w), jnp.int32),
        pltpu.VMEM((NBUF, hcols, d), jnp.float32),
    ] + [pltpu.SemaphoreType.DMA] * (2 * NBUF)

    @functools.partial(
        pl.kernel,
        mesh=mesh,
        out_type=jax.ShapeDtypeStruct((seq, n_rows, d), jnp.float32),
        scratch_types=scratch,
    )
    def emb(xt_hbm, w_hbm, out_hbm, idx_v, rows_v, *sems):
        gsems, osems = sems[:NBUF], sems[NBUF:]
        wid = lax.axis_index("s") * NC + lax.axis_index("c")
        col0 = wid * cols_per_w
        pltpu.sync_copy(xt_hbm.at[:, pl.ds(col0, cols_per_w)], idx_v)

        def gather(c, b):
            j, h = c // HSPLIT, c % HSPLIT
            return pltpu.make_async_copy(
                w_hbm.at[idx_v.at[j, pl.ds(h * hcols, hcols)]], rows_v.at[b], gsems[b]
            )

        def write(c, b):
            j, h = c // HSPLIT, c % HSPLIT
            return pltpu.make_async_copy(
                rows_v.at[b], out_hbm.at[j, pl.ds(col0 + h * hcols, hcols)], osems[b]
            )

        def outer(co, carry):
            for b in range(NBUF):
                c = co * NBUF + b

                @pl.when(c >= NBUF)
                def _():
                    write(c - NBUF, b).wait()

                gather(c, b).start()

                cc = c - (NBUF - 1)
                bc = (b + 1) % NBUF

                @pl.when(cc >= 0)
                def _():
                    gather(cc, bc).wait()
                    write(cc, bc).start()

            return carry

        lax.fori_loop(0, nchunk // NBUF, outer, 0)

        for k in range(NBUF - 1):
            cc = nchunk - (NBUF - 1) + k
            gather(cc, cc % NBUF).wait()
            write(cc, cc % NBUF).start()
        for b in range(NBUF):
            write(nchunk - NBUF + b, b).wait()

    return emb


def kernel(x, weight):
    n_rows, seq = x.shape
    vocab, d = weight.shape
    out_t = _make_emb(seq, n_rows, vocab, d)(x.T, weight)
    return out_t.transpose(1, 0, 2)
